# baseline (speedup 1.0000x reference)
import functools

import jax
import jax.numpy as jnp
from jax import lax
from jax.experimental import pallas as pl
from jax.experimental.pallas import tpu as pltpu
from jax.experimental.pallas import tpu_sc as plsc

N_NODES = 100000
N_RELS = 1000
EMB = 200
B = 16384

ROW = EMB * 2
NVEC = ROW // 16
NC = 2
NS = 16
NW = NC * NS
PER_W = B // NW
CHUNK = 64
NCHUNK = PER_W // CHUNK
RELW = 64

_GDN = lax.GatherDimensionNumbers(
    offset_dims=(), collapsed_slice_dims=(0,), start_index_map=(0,))


def _perm(x, idx):
    return lax.gather(x, idx[:, None], dimension_numbers=_GDN,
                      slice_sizes=(1,),
                      mode=lax.GatherScatterMode.PROMISE_IN_BOUNDS)


def _rsqrt(x):
    xi = lax.bitcast_convert_type(x, jnp.int32)
    yi = jnp.int32(0x5F3759DF) - (xi >> 1)
    y = lax.bitcast_convert_type(yi, jnp.float32)
    hx = x * jnp.float32(0.5)
    for _ in range(3):
        y = y * (jnp.float32(1.5) - hx * y * y)
    return y


def _merge(a, b, s, lane):
    pa = _perm(a, lane ^ s)
    pb = _perm(b, lane ^ s)
    return jnp.where((lane & s) == 0, a + pa, b + pb)


@functools.partial(
    pl.kernel,
    mesh=plsc.VectorSubcoreMesh(core_axis_name="c", subcore_axis_name="s"),
    out_type=jax.ShapeDtypeStruct((B,), jnp.float32),
    compiler_params=pltpu.CompilerParams(needs_layout_passes=False),
    scratch_types=[
        pltpu.VMEM((CHUNK,), jnp.int32),
        pltpu.VMEM((CHUNK,), jnp.int32),
        pltpu.VMEM((CHUNK,), jnp.int32),
        pltpu.VMEM((CHUNK, ROW), jnp.float32),
        pltpu.VMEM((CHUNK * ROW,), jnp.float32),
        pltpu.VMEM((CHUNK, ROW), jnp.float32),
        pltpu.VMEM((ROW,), jnp.float32),
        pltpu.VMEM((CHUNK * 16,), jnp.float32),
        pltpu.VMEM((PER_W,), jnp.float32),
        pltpu.VMEM_SHARED((N_RELS * 512,), jnp.float32),
        pltpu.SemaphoreType.DMA,
        pltpu.SemaphoreType.DMA,
    ],
)
def _rotate_score(s_idx_hbm, r_idx_hbm, o_idx_hbm, ent_hbm, rel_hbm,
                  out_hbm, s_iv, r_iv, o_iv,
                  s_rows, r_flat, o_rows, stage, accbuf, scores, rel_sh,
                  sem, sem2):
    lane = lax.iota(jnp.int32, 16)
    swap_idx = lane ^ 1
    even_idx = lane & jnp.int32(-2)
    odd_idx = lane | jnp.int32(1)
    altsign = jnp.where((lane & 1) == 0, jnp.float32(-1.0), jnp.float32(1.0))

    cid = lax.axis_index("c")
    sid = lax.axis_index("s")
    w = sid * NC + cid
    base = w * PER_W

    def normalize(start, nrows):
        def fire1(rr, c2):
            pltpu.async_copy(rel_hbm.at[start + rr], s_rows.at[rr], sem)
            return c2
        lax.fori_loop(0, nrows, fire1, jnp.int32(0))

        def drain1(rr, c2):
            pltpu.make_async_copy(rel_hbm.at[0], s_rows.at[rr], sem).wait()
            return c2
        lax.fori_loop(0, nrows, drain1, jnp.int32(0))

        def nbody(rr, c2):
            for j in range(NVEC):
                rv = s_rows[rr, pl.ds(j * 16, 16)]
                sw = _perm(rv, swap_idx)
                abs2 = rv * rv + sw * sw
                inv = jnp.minimum(_rsqrt(abs2), jnp.float32(1e9))
                stage[pl.ds(j * 16, 16)] = rv * inv
            pltpu.sync_copy(stage,
                            rel_sh.at[pl.ds((start + rr) * 512, ROW)])
            return c2
        lax.fori_loop(0, nrows, nbody, jnp.int32(0))

    @pl.when(sid < NS - 1)
    def _():
        normalize(sid * RELW, RELW)

    @pl.when(sid == NS - 1)
    def _():
        normalize((NS - 1) * RELW, N_RELS - (NS - 1) * RELW)

    plsc.subcore_barrier()

    def chunk_body(c, carry):
        cb = base + c * CHUNK
        pltpu.sync_copy(s_idx_hbm.at[pl.ds(cb, CHUNK)], s_iv)
        pltpu.sync_copy(r_idx_hbm.at[pl.ds(cb, CHUNK)], r_iv)
        pltpu.sync_copy(o_idx_hbm.at[pl.ds(cb, CHUNK)], o_iv)

        def fire(g, carry2):
            sv16 = s_iv[pl.ds(g * 16, 16)]
            rv16 = r_iv[pl.ds(g * 16, 16)]
            ov16 = o_iv[pl.ds(g * 16, 16)]
            e0 = g * 16
            for l in range(16):
                e = e0 + l
                pltpu.async_copy(ent_hbm.at[sv16[l]], s_rows.at[e], sem)
                pltpu.async_copy(rel_sh.at[pl.ds(rv16[l] * 512, ROW)],
                                 r_flat.at[pl.ds(e * ROW, ROW)], sem2)
                pltpu.async_copy(ent_hbm.at[ov16[l]], o_rows.at[e], sem)
            return carry2
        lax.fori_loop(0, CHUNK // 16, fire, jnp.int32(0))

        def drain(e, carry2):
            pltpu.make_async_copy(ent_hbm.at[0], s_rows.at[e], sem).wait()
            pltpu.make_async_copy(rel_sh.at[pl.ds(0, ROW)],
                                  r_flat.at[pl.ds(e * ROW, ROW)], sem2).wait()
            pltpu.make_async_copy(ent_hbm.at[0], o_rows.at[e], sem).wait()
            return carry2
        lax.fori_loop(0, CHUNK, drain, jnp.int32(0))

        def body(e, carry2):
            eb = e * ROW
            acc = jnp.zeros((16,), jnp.float32)
            for j in range(NVEC):
                sv = s_rows[e, pl.ds(j * 16, 16)]
                rv = r_flat[pl.ds(eb + j * 16, 16)]
                ov = o_rows[e, pl.ds(j * 16, 16)]
                ssw = _perm(sv, swap_idx)
                ra = _perm(rv, even_idx)
                rb = _perm(rv, odd_idx) * altsign
                rot = sv * ra + ssw * rb
                d = rot - ov
                acc = acc + d * d
            accbuf[pl.ds(e * 16, 16)] = acc
            return carry2
        lax.fori_loop(0, CHUNK, body, jnp.int32(0))

        def reduce_body(g, carry2):
            gb = g * 256
            vs = [accbuf[pl.ds(gb + i * 16, 16)] for i in range(16)]
            for s in (1, 2, 4, 8):
                vs = [_merge(vs[i], vs[i + 1], s, lane)
                      for i in range(0, len(vs), 2)]
            tot = vs[0]
            y = _rsqrt(jnp.maximum(tot, jnp.float32(1e-38)))
            scores[pl.ds(c * CHUNK + g * 16, 16)] = tot * y
            return carry2
        lax.fori_loop(0, CHUNK // 16, reduce_body, jnp.int32(0))
        return carry

    lax.fori_loop(0, NCHUNK, chunk_body, jnp.int32(0))
    pltpu.sync_copy(scores, out_hbm.at[pl.ds(base, PER_W)])


def kernel(s_idx, r_idx, o_idx, ent, rel):
    s_idx = s_idx.astype(jnp.int32)
    r_idx = r_idx.astype(jnp.int32)
    o_idx = o_idx.astype(jnp.int32)
    return _rotate_score(s_idx, r_idx, o_idx, ent, rel)


# speedup vs baseline: 1.0746x; 1.0746x over previous
import functools

import jax
import jax.numpy as jnp
from jax import lax
from jax.experimental import pallas as pl
from jax.experimental.pallas import tpu as pltpu
from jax.experimental.pallas import tpu_sc as plsc

N_NODES = 100000
N_RELS = 1000
EMB = 200
B = 16384

ROW = EMB * 2
NVEC = ROW // 16
NC = 2
NS = 16
NW = NC * NS
PER_W = B // NW
CHUNK = 32
NCHUNK = PER_W // CHUNK

_GDN = lax.GatherDimensionNumbers(
    offset_dims=(), collapsed_slice_dims=(0,), start_index_map=(0,))


def _perm(x, idx):
    return lax.gather(x, idx[:, None], dimension_numbers=_GDN,
                      slice_sizes=(1,),
                      mode=lax.GatherScatterMode.PROMISE_IN_BOUNDS)


def _rsqrt(x):
    xi = lax.bitcast_convert_type(x, jnp.int32)
    yi = jnp.int32(0x5F3759DF) - (xi >> 1)
    y = lax.bitcast_convert_type(yi, jnp.float32)
    hx = x * jnp.float32(0.5)
    for _ in range(3):
        y = y * (jnp.float32(1.5) - hx * y * y)
    return y


def _merge(a, b, s, lane):
    pa = _perm(a, lane ^ s)
    pb = _perm(b, lane ^ s)
    return jnp.where((lane & s) == 0, a + pa, b + pb)


def _mesh():
    return plsc.VectorSubcoreMesh(core_axis_name="c", subcore_axis_name="s")


def _worker_id():
    return lax.axis_index("s") * NC + lax.axis_index("c")


@functools.partial(
    pl.kernel,
    mesh=_mesh(),
    out_type=jax.ShapeDtypeStruct((N_RELS, ROW), jnp.float32),
    compiler_params=pltpu.CompilerParams(use_tc_tiling_on_sc=False),
    scratch_types=[pltpu.VMEM((32, ROW), jnp.float32)],
)
def _normalize_rel(rel_hbm, out_hbm, buf):
    lane = lax.iota(jnp.int32, 16)
    swap_idx = lane ^ 1
    w = _worker_id()
    tail = N_RELS - 31 * 32

    def process(nrows):
        def body(r, carry):
            for j in range(NVEC):
                rv = buf[r, pl.ds(j * 16, 16)]
                sw = _perm(rv, swap_idx)
                abs2 = rv * rv + sw * sw
                inv = jnp.minimum(_rsqrt(abs2), jnp.float32(1e9))
                buf[r, pl.ds(j * 16, 16)] = rv * inv
            return carry
        lax.fori_loop(0, nrows, body, jnp.int32(0))

    @pl.when(w < 31)
    def _():
        pltpu.sync_copy(rel_hbm.at[pl.ds(w * 32, 32)], buf)
        process(32)
        pltpu.sync_copy(buf, out_hbm.at[pl.ds(w * 32, 32)])

    @pl.when(w == 31)
    def _():
        pltpu.sync_copy(rel_hbm.at[pl.ds(31 * 32, tail)], buf.at[pl.ds(0, tail)])
        process(tail)
        pltpu.sync_copy(buf.at[pl.ds(0, tail)], out_hbm.at[pl.ds(31 * 32, tail)])


@functools.partial(
    pl.kernel,
    mesh=_mesh(),
    out_type=jax.ShapeDtypeStruct((B,), jnp.float32),
    compiler_params=pltpu.CompilerParams(needs_layout_passes=False),
    scratch_types=[
        pltpu.VMEM((CHUNK,), jnp.int32),
        pltpu.VMEM((CHUNK,), jnp.int32),
        pltpu.VMEM((CHUNK,), jnp.int32),
        pltpu.VMEM((CHUNK,), jnp.int32),
        pltpu.VMEM((CHUNK,), jnp.int32),
        pltpu.VMEM((CHUNK,), jnp.int32),
        pltpu.VMEM((2 * CHUNK, ROW), jnp.float32),
        pltpu.VMEM((2 * CHUNK, ROW), jnp.float32),
        pltpu.VMEM((2 * CHUNK, ROW), jnp.float32),
        pltpu.VMEM((CHUNK * 16,), jnp.float32),
        pltpu.VMEM((PER_W,), jnp.float32),
        pltpu.SemaphoreType.DMA,
        pltpu.SemaphoreType.DMA,
    ],
)
def _rotate_score(s_idx_hbm, r_idx_hbm, o_idx_hbm, ent_hbm, reln_hbm,
                  out_hbm, s_iv0, r_iv0, o_iv0, s_iv1, r_iv1, o_iv1,
                  s_rows, r_rows, o_rows,
                  accbuf, scores, semA, semB):
    set0 = (s_iv0, r_iv0, o_iv0, 0)
    set1 = (s_iv1, r_iv1, o_iv1, CHUNK)
    lane = lax.iota(jnp.int32, 16)
    swap_idx = lane ^ 1
    even_idx = lane & jnp.int32(-2)
    odd_idx = lane | jnp.int32(1)
    altsign = jnp.where((lane & 1) == 0, jnp.float32(-1.0), jnp.float32(1.0))

    w = _worker_id()
    base = w * PER_W

    def stage_and_fire(c, bufset, sem):
        s_iv, r_iv, o_iv, off = bufset
        cb = base + c * CHUNK
        pltpu.sync_copy(s_idx_hbm.at[pl.ds(cb, CHUNK)], s_iv)
        pltpu.sync_copy(r_idx_hbm.at[pl.ds(cb, CHUNK)], r_iv)
        pltpu.sync_copy(o_idx_hbm.at[pl.ds(cb, CHUNK)], o_iv)

        def fire(g, carry2):
            sv16 = s_iv[pl.ds(g * 16, 16)]
            rv16 = r_iv[pl.ds(g * 16, 16)]
            ov16 = o_iv[pl.ds(g * 16, 16)]
            e0 = off + g * 16
            for l in range(16):
                pltpu.async_copy(ent_hbm.at[sv16[l]], s_rows.at[e0 + l], sem)
                pltpu.async_copy(reln_hbm.at[rv16[l]], r_rows.at[e0 + l], sem)
                pltpu.async_copy(ent_hbm.at[ov16[l]], o_rows.at[e0 + l], sem)
            return carry2
        lax.fori_loop(0, CHUNK // 16, fire, jnp.int32(0))

    def drain_and_compute(c, bufset, sem):
        _, _, _, off = bufset
        def drain(e, carry2):
            pltpu.make_async_copy(ent_hbm.at[0], s_rows.at[off + e], sem).wait()
            pltpu.make_async_copy(reln_hbm.at[0], r_rows.at[off + e], sem).wait()
            pltpu.make_async_copy(ent_hbm.at[0], o_rows.at[off + e], sem).wait()
            return carry2
        lax.fori_loop(0, CHUNK, drain, jnp.int32(0))

        def body(eb, carry2):
            e = off + eb
            acc = jnp.zeros((16,), jnp.float32)
            for j in range(NVEC):
                sv = s_rows[e, pl.ds(j * 16, 16)]
                rv = r_rows[e, pl.ds(j * 16, 16)]
                ov = o_rows[e, pl.ds(j * 16, 16)]
                ssw = _perm(sv, swap_idx)
                ra = _perm(rv, even_idx)
                rb = _perm(rv, odd_idx) * altsign
                rot = sv * ra + ssw * rb
                d = rot - ov
                acc = acc + d * d
            accbuf[pl.ds(eb * 16, 16)] = acc
            return carry2
        lax.fori_loop(0, CHUNK, body, jnp.int32(0))

        def reduce_body(g, carry2):
            gb = g * 256
            vs = [accbuf[pl.ds(gb + i * 16, 16)] for i in range(16)]
            for s in (1, 2, 4, 8):
                vs = [_merge(vs[i], vs[i + 1], s, lane)
                      for i in range(0, len(vs), 2)]
            tot = vs[0]
            y = _rsqrt(jnp.maximum(tot, jnp.float32(1e-38)))
            scores[pl.ds(c * CHUNK + g * 16, 16)] = tot * y
            return carry2
        lax.fori_loop(0, CHUNK // 16, reduce_body, jnp.int32(0))

    stage_and_fire(jnp.int32(0), set0, semA)

    def pair_body(k, carry):
        c0 = k * 2
        stage_and_fire(c0 + 1, set1, semB)
        drain_and_compute(c0, set0, semA)
        stage_and_fire((c0 + 2) & (NCHUNK - 1), set0, semA)
        drain_and_compute(c0 + 1, set1, semB)
        return carry
    lax.fori_loop(0, NCHUNK // 2, pair_body, jnp.int32(0))

    def final_drain(e, carry):
        pltpu.make_async_copy(ent_hbm.at[0], s_rows.at[e], semA).wait()
        pltpu.make_async_copy(reln_hbm.at[0], r_rows.at[e], semA).wait()
        pltpu.make_async_copy(ent_hbm.at[0], o_rows.at[e], semA).wait()
        return carry
    lax.fori_loop(0, CHUNK, final_drain, jnp.int32(0))

    pltpu.sync_copy(scores, out_hbm.at[pl.ds(base, PER_W)])


def kernel(s_idx, r_idx, o_idx, ent, rel):
    s_idx = s_idx.astype(jnp.int32)
    r_idx = r_idx.astype(jnp.int32)
    o_idx = o_idx.astype(jnp.int32)
    rel_n = _normalize_rel(rel)
    return _rotate_score(s_idx, r_idx, o_idx, ent, rel_n)


# speedup vs baseline: 1.0961x; 1.0200x over previous
import functools

import jax
import jax.numpy as jnp
from jax import lax
from jax.experimental import pallas as pl
from jax.experimental.pallas import tpu as pltpu
from jax.experimental.pallas import tpu_sc as plsc

N_NODES = 100000
N_RELS = 1000
EMB = 200
B = 16384

ROW = EMB * 2
NVEC = ROW // 16
NC = 2
NS = 16
NW = NC * NS
PER_W = B // NW
CHUNK = 64
NCHUNK = PER_W // CHUNK

_GDN = lax.GatherDimensionNumbers(
    offset_dims=(), collapsed_slice_dims=(0,), start_index_map=(0,))


def _perm(x, idx):
    return lax.gather(x, idx[:, None], dimension_numbers=_GDN,
                      slice_sizes=(1,),
                      mode=lax.GatherScatterMode.PROMISE_IN_BOUNDS)


def _rsqrt(x):
    xi = lax.bitcast_convert_type(x, jnp.int32)
    yi = jnp.int32(0x5F3759DF) - (xi >> 1)
    y = lax.bitcast_convert_type(yi, jnp.float32)
    hx = x * jnp.float32(0.5)
    for _ in range(3):
        y = y * (jnp.float32(1.5) - hx * y * y)
    return y


def _merge(a, b, s, lane):
    pa = _perm(a, lane ^ s)
    pb = _perm(b, lane ^ s)
    return jnp.where((lane & s) == 0, a + pa, b + pb)


def _mesh():
    return plsc.VectorSubcoreMesh(core_axis_name="c", subcore_axis_name="s")


def _worker_id():
    return lax.axis_index("s") * NC + lax.axis_index("c")


@functools.partial(
    pl.kernel,
    mesh=_mesh(),
    out_type=jax.ShapeDtypeStruct((N_RELS, ROW), jnp.float32),
    compiler_params=pltpu.CompilerParams(use_tc_tiling_on_sc=False),
    scratch_types=[pltpu.VMEM((32, ROW), jnp.float32)],
)
def _normalize_rel(rel_hbm, out_hbm, buf):
    lane = lax.iota(jnp.int32, 16)
    swap_idx = lane ^ 1
    w = _worker_id()
    tail = N_RELS - 31 * 32

    def process(nrows):
        def body(r, carry):
            for j in range(NVEC):
                rv = buf[r, pl.ds(j * 16, 16)]
                sw = _perm(rv, swap_idx)
                abs2 = rv * rv + sw * sw
                inv = jnp.minimum(_rsqrt(abs2), jnp.float32(1e9))
                buf[r, pl.ds(j * 16, 16)] = rv * inv
            return carry
        lax.fori_loop(0, nrows, body, jnp.int32(0))

    @pl.when(w < 31)
    def _():
        pltpu.sync_copy(rel_hbm.at[pl.ds(w * 32, 32)], buf)
        process(32)
        pltpu.sync_copy(buf, out_hbm.at[pl.ds(w * 32, 32)])

    @pl.when(w == 31)
    def _():
        pltpu.sync_copy(rel_hbm.at[pl.ds(31 * 32, tail)], buf.at[pl.ds(0, tail)])
        process(tail)
        pltpu.sync_copy(buf.at[pl.ds(0, tail)], out_hbm.at[pl.ds(31 * 32, tail)])


@functools.partial(
    pl.kernel,
    mesh=_mesh(),
    out_type=jax.ShapeDtypeStruct((B,), jnp.float32),
    compiler_params=pltpu.CompilerParams(needs_layout_passes=False),
    scratch_types=[
        pltpu.VMEM((CHUNK,), jnp.int32),
        pltpu.VMEM((CHUNK,), jnp.int32),
        pltpu.VMEM((CHUNK,), jnp.int32),
        pltpu.VMEM((CHUNK, ROW), jnp.float32),
        pltpu.VMEM((CHUNK, ROW), jnp.float32),
        pltpu.VMEM((CHUNK, ROW), jnp.float32),
        pltpu.VMEM((CHUNK * 16,), jnp.float32),
        pltpu.VMEM((PER_W,), jnp.float32),
        pltpu.SemaphoreType.DMA,
    ],
)
def _rotate_score(s_idx_hbm, r_idx_hbm, o_idx_hbm, ent_hbm, reln_hbm,
                  out_hbm, s_iv, r_iv, o_iv,
                  s_rows, r_rows, o_rows, accbuf, scores, sem):
    lane = lax.iota(jnp.int32, 16)
    swap_idx = lane ^ 1
    even_idx = lane & jnp.int32(-2)
    odd_idx = lane | jnp.int32(1)
    altsign = jnp.where((lane & 1) == 0, jnp.float32(-1.0), jnp.float32(1.0))

    w = _worker_id()
    base = w * PER_W

    def chunk_body(c, carry):
        cb = base + c * CHUNK
        pltpu.sync_copy(s_idx_hbm.at[pl.ds(cb, CHUNK)], s_iv)
        pltpu.sync_copy(r_idx_hbm.at[pl.ds(cb, CHUNK)], r_iv)
        pltpu.sync_copy(o_idx_hbm.at[pl.ds(cb, CHUNK)], o_iv)

        def fire(g, carry2):
            sv16 = s_iv[pl.ds(g * 16, 16)]
            rv16 = r_iv[pl.ds(g * 16, 16)]
            ov16 = o_iv[pl.ds(g * 16, 16)]
            e0 = g * 16
            for l in range(16):
                pltpu.async_copy(ent_hbm.at[sv16[l]], s_rows.at[e0 + l], sem)
                pltpu.async_copy(reln_hbm.at[rv16[l]], r_rows.at[e0 + l], sem)
                pltpu.async_copy(ent_hbm.at[ov16[l]], o_rows.at[e0 + l], sem)
            return carry2
        lax.fori_loop(0, CHUNK // 16, fire, jnp.int32(0))

        def drain(e, carry2):
            pltpu.make_async_copy(ent_hbm.at[0], s_rows.at[e], sem).wait()
            pltpu.make_async_copy(ent_hbm.at[0], r_rows.at[e], sem).wait()
            pltpu.make_async_copy(ent_hbm.at[0], o_rows.at[e], sem).wait()
            return carry2
        lax.fori_loop(0, CHUNK, drain, jnp.int32(0))

        def body(e, carry2):
            acc = jnp.zeros((16,), jnp.float32)
            for j in range(NVEC):
                sv = s_rows[e, pl.ds(j * 16, 16)]
                rv = r_rows[e, pl.ds(j * 16, 16)]
                ov = o_rows[e, pl.ds(j * 16, 16)]
                ssw = _perm(sv, swap_idx)
                ra = _perm(rv, even_idx)
                rb = _perm(rv, odd_idx) * altsign
                rot = sv * ra + ssw * rb
                d = rot - ov
                acc = acc + d * d
            accbuf[pl.ds(e * 16, 16)] = acc
            return carry2
        lax.fori_loop(0, CHUNK, body, jnp.int32(0))

        def reduce_body(g, carry2):
            gb = g * 256
            vs = [accbuf[pl.ds(gb + i * 16, 16)] for i in range(16)]
            for s in (1, 2, 4, 8):
                vs = [_merge(vs[i], vs[i + 1], s, lane)
                      for i in range(0, len(vs), 2)]
            tot = vs[0]
            y = _rsqrt(jnp.maximum(tot, jnp.float32(1e-38)))
            scores[pl.ds(c * CHUNK + g * 16, 16)] = tot * y
            return carry2
        lax.fori_loop(0, CHUNK // 16, reduce_body, jnp.int32(0))
        return carry

    lax.fori_loop(0, NCHUNK, chunk_body, jnp.int32(0))
    pltpu.sync_copy(scores, out_hbm.at[pl.ds(base, PER_W)])


def kernel(s_idx, r_idx, o_idx, ent, rel):
    s_idx = s_idx.astype(jnp.int32)
    r_idx = r_idx.astype(jnp.int32)
    o_idx = o_idx.astype(jnp.int32)
    rel_n = _normalize_rel(rel)
    return _rotate_score(s_idx, r_idx, o_idx, ent, rel_n)


# speedup vs baseline: 1.0965x; 1.0004x over previous
import functools

import jax
import jax.numpy as jnp
from jax import lax
from jax.experimental import pallas as pl
from jax.experimental.pallas import tpu as pltpu
from jax.experimental.pallas import tpu_sc as plsc

N_NODES = 100000
N_RELS = 1000
EMB = 200
B = 16384

ROW = EMB * 2
NVEC = ROW // 16
NC = 2
NS = 16
NW = NC * NS
PER_W = B // NW
CHUNK = 64
NCHUNK = PER_W // CHUNK

_GDN = lax.GatherDimensionNumbers(
    offset_dims=(), collapsed_slice_dims=(0,), start_index_map=(0,))


def _perm(x, idx):
    return lax.gather(x, idx[:, None], dimension_numbers=_GDN,
                      slice_sizes=(1,),
                      mode=lax.GatherScatterMode.PROMISE_IN_BOUNDS)


def _rsqrt(x):
    xi = lax.bitcast_convert_type(x, jnp.int32)
    yi = jnp.int32(0x5F3759DF) - (xi >> 1)
    y = lax.bitcast_convert_type(yi, jnp.float32)
    hx = x * jnp.float32(0.5)
    for _ in range(3):
        y = y * (jnp.float32(1.5) - hx * y * y)
    return y


def _merge(a, b, s, lane):
    pa = _perm(a, lane ^ s)
    pb = _perm(b, lane ^ s)
    return jnp.where((lane & s) == 0, a + pa, b + pb)


def _mesh():
    return plsc.VectorSubcoreMesh(core_axis_name="c", subcore_axis_name="s")


def _worker_id():
    return lax.axis_index("s") * NC + lax.axis_index("c")


@functools.partial(
    pl.kernel,
    mesh=_mesh(),
    out_type=jax.ShapeDtypeStruct((N_RELS, ROW), jnp.float32),
    compiler_params=pltpu.CompilerParams(use_tc_tiling_on_sc=False),
    scratch_types=[pltpu.VMEM((32, ROW), jnp.float32)],
)
def _normalize_rel(rel_hbm, out_hbm, buf):
    lane = lax.iota(jnp.int32, 16)
    swap_idx = lane ^ 1
    w = _worker_id()
    tail = N_RELS - 31 * 32

    def process(nrows):
        def body(r, carry):
            for j in range(NVEC):
                rv = buf[r, pl.ds(j * 16, 16)]
                sw = _perm(rv, swap_idx)
                abs2 = rv * rv + sw * sw
                inv = jnp.minimum(_rsqrt(abs2), jnp.float32(1e9))
                buf[r, pl.ds(j * 16, 16)] = rv * inv
            return carry
        lax.fori_loop(0, nrows, body, jnp.int32(0))

    @pl.when(w < 31)
    def _():
        pltpu.sync_copy(rel_hbm.at[pl.ds(w * 32, 32)], buf)
        process(32)
        pltpu.sync_copy(buf, out_hbm.at[pl.ds(w * 32, 32)])

    @pl.when(w == 31)
    def _():
        pltpu.sync_copy(rel_hbm.at[pl.ds(31 * 32, tail)], buf.at[pl.ds(0, tail)])
        process(tail)
        pltpu.sync_copy(buf.at[pl.ds(0, tail)], out_hbm.at[pl.ds(31 * 32, tail)])


@functools.partial(
    pl.kernel,
    mesh=_mesh(),
    out_type=jax.ShapeDtypeStruct((B,), jnp.float32),
    compiler_params=pltpu.CompilerParams(needs_layout_passes=False),
    scratch_types=[
        pltpu.VMEM((CHUNK,), jnp.int32),
        pltpu.VMEM((CHUNK,), jnp.int32),
        pltpu.VMEM((CHUNK,), jnp.int32),
        pltpu.VMEM((CHUNK, ROW), jnp.float32),
        pltpu.VMEM((CHUNK, ROW), jnp.float32),
        pltpu.VMEM((CHUNK, ROW), jnp.float32),
        pltpu.VMEM((CHUNK * 16,), jnp.float32),
        pltpu.VMEM((PER_W,), jnp.float32),
        pltpu.SemaphoreType.DMA,
    ],
)
def _rotate_score(s_idx_hbm, r_idx_hbm, o_idx_hbm, ent_hbm, reln_hbm,
                  out_hbm, s_iv, r_iv, o_iv,
                  s_rows, r_rows, o_rows, accbuf, scores, sem):
    lane = lax.iota(jnp.int32, 16)
    swap_idx = lane ^ 1
    even_idx = lane & jnp.int32(-2)
    odd_idx = lane | jnp.int32(1)
    altsign = jnp.where((lane & 1) == 0, jnp.float32(-1.0), jnp.float32(1.0))

    w = _worker_id()
    base = w * PER_W

    def chunk_body(c, carry):
        cb = base + c * CHUNK
        pltpu.sync_copy(s_idx_hbm.at[pl.ds(cb, CHUNK)], s_iv)
        pltpu.sync_copy(r_idx_hbm.at[pl.ds(cb, CHUNK)], r_iv)
        pltpu.sync_copy(o_idx_hbm.at[pl.ds(cb, CHUNK)], o_iv)

        def fire(g, carry2):
            sv16 = s_iv[pl.ds(g * 16, 16)]
            rv16 = r_iv[pl.ds(g * 16, 16)]
            ov16 = o_iv[pl.ds(g * 16, 16)]
            e0 = g * 16
            for l in range(16):
                pltpu.async_copy(ent_hbm.at[sv16[l]], s_rows.at[e0 + l], sem)
                pltpu.async_copy(reln_hbm.at[rv16[l]], r_rows.at[e0 + l], sem)
                pltpu.async_copy(ent_hbm.at[ov16[l]], o_rows.at[e0 + l], sem)
            return carry2
        lax.fori_loop(0, CHUNK // 16, fire, jnp.int32(0))

        pltpu.make_async_copy(ent_hbm.at[pl.ds(0, CHUNK)], s_rows, sem).wait()
        pltpu.make_async_copy(ent_hbm.at[pl.ds(0, CHUNK)], r_rows, sem).wait()
        pltpu.make_async_copy(ent_hbm.at[pl.ds(0, CHUNK)], o_rows, sem).wait()

        def body(e, carry2):
            acc = jnp.zeros((16,), jnp.float32)
            for j in range(NVEC):
                sv = s_rows[e, pl.ds(j * 16, 16)]
                rv = r_rows[e, pl.ds(j * 16, 16)]
                ov = o_rows[e, pl.ds(j * 16, 16)]
                ssw = _perm(sv, swap_idx)
                ra = _perm(rv, even_idx)
                rb = _perm(rv, odd_idx) * altsign
                rot = sv * ra + ssw * rb
                d = rot - ov
                acc = acc + d * d
            accbuf[pl.ds(e * 16, 16)] = acc
            return carry2
        lax.fori_loop(0, CHUNK, body, jnp.int32(0))

        def reduce_body(g, carry2):
            gb = g * 256
            vs = [accbuf[pl.ds(gb + i * 16, 16)] for i in range(16)]
            for s in (1, 2, 4, 8):
                vs = [_merge(vs[i], vs[i + 1], s, lane)
                      for i in range(0, len(vs), 2)]
            tot = vs[0]
            y = _rsqrt(jnp.maximum(tot, jnp.float32(1e-38)))
            scores[pl.ds(c * CHUNK + g * 16, 16)] = tot * y
            return carry2
        lax.fori_loop(0, CHUNK // 16, reduce_body, jnp.int32(0))
        return carry

    lax.fori_loop(0, NCHUNK, chunk_body, jnp.int32(0))
    pltpu.sync_copy(scores, out_hbm.at[pl.ds(base, PER_W)])


def kernel(s_idx, r_idx, o_idx, ent, rel):
    s_idx = s_idx.astype(jnp.int32)
    r_idx = r_idx.astype(jnp.int32)
    o_idx = o_idx.astype(jnp.int32)
    rel_n = _normalize_rel(rel)
    return _rotate_score(s_idx, r_idx, o_idx, ent, rel_n)
